# baseline (speedup 1.0000x reference)
import jax
import jax.numpy as jnp
from jax.experimental import pallas as pl


def _body(x_ref, pe_ref, out_ref):
    pe_t = jnp.transpose(pe_ref[...], (1, 0))
    out_ref[...] = x_ref[...] + pe_t[None, :, :]


def kernel(x, pe_table):
    b, d, s = x.shape
    s_blk = 1024
    b_blk = 2
    grid = (s // s_blk, b // b_blk)
    return pl.pallas_call(
        _body,
        grid=grid,
        in_specs=[
            pl.BlockSpec((b_blk, d, s_blk), lambda si, bi: (bi, 0, si)),
            pl.BlockSpec((s_blk, d), lambda si, bi: (si, 0)),
        ],
        out_specs=pl.BlockSpec((b_blk, d, s_blk), lambda si, bi: (bi, 0, si)),
        out_shape=jax.ShapeDtypeStruct((b, d, s), x.dtype),
    )(x, pe_table)
